# baseline (speedup 1.0000x reference)
import functools

import jax
import jax.numpy as jnp
from jax import lax
from jax.experimental import pallas as pl
from jax.experimental.pallas import tpu as pltpu
from jax.experimental.pallas import tpu_sc as plsc

_N = 10000
_E = 320000
_F = 128
_NORM_SCALE = 1.0

_NC = 2
_NS = 16
_NW = _NC * _NS
_EW = _E // _NW
_C = 80
_ECP = 10240
_NCH = 126
_NP = 10240
_RT = _NP // _NS

_mesh = plsc.VectorSubcoreMesh(core_axis_name="c", subcore_axis_name="s")


@functools.partial(
    pl.kernel,
    mesh=_mesh,
    out_type=jax.ShapeDtypeStruct((_NC, _NP, _F), jnp.float32),
    scratch_types=(
        [pltpu.VMEM((_C, _F), jnp.float32) for _ in range(2)]
        + [pltpu.VMEM((_C,), jnp.int32) for _ in range(4)]
        + [pltpu.VMEM((_C,), jnp.float32) for _ in range(2)]
        + [pltpu.VMEM_SHARED((_NP, _F), jnp.float32)]
        + [pltpu.SemaphoreType.DMA for _ in range(9)]
    ),
)
def _spmm(h_hbm, src_hbm, dst_hbm, vals_hbm, out_hbm,
          rowsA, rowsB, srcA, srcB, dstA, dstB, valsA, valsB, acc_sh,
          gsem, ssA, ssB, isA0, isA1, isA2, isB0, isB1, isB2):
    c = lax.axis_index("c")
    s = lax.axis_index("s")
    wid = c * _NS + s

    zvec = jnp.zeros((16,), jnp.float32)

    def zrow(r, carry):
        for j in range(_F // 16):
            rowsA[r, pl.ds(j * 16, 16)] = zvec
        return carry

    lax.fori_loop(0, _C, zrow, 0)
    for k in range(_RT // _C):
        pltpu.sync_copy(rowsA, acc_sh.at[pl.ds(s * _RT + k * _C, _C)])
    plsc.subcore_barrier()

    def scale(buf, vv):
        def group(g, gcarry):
            v16 = vv[pl.ds(g * 16, 16)]
            for i in range(16):
                vvec = v16[jnp.full((16,), i, jnp.int32)]
                r = g * 16 + i
                for j in range(_F // 16):
                    seg = buf[r, pl.ds(j * 16, 16)]
                    buf[r, pl.ds(j * 16, 16)] = seg * vvec
            return gcarry

        lax.fori_loop(0, _C // 16, group, 0)

    sets = ((srcA, dstA, valsA, isA0, isA1, isA2),
            (srcB, dstB, valsB, isB0, isB1, isB2))

    def issue_idx(m, st):
        sv, dv, vv, s0, s1, s2 = st
        base = wid * _ECP + m * _C
        pltpu.async_copy(src_hbm.at[pl.ds(base, _C)], sv, s0)
        pltpu.async_copy(dst_hbm.at[pl.ds(base, _C)], dv, s1)
        pltpu.async_copy(vals_hbm.at[pl.ds(base, _C)], vv, s2)

    def wait_idx(m, st):
        sv, dv, vv, s0, s1, s2 = st
        base = wid * _ECP + m * _C
        pltpu.make_async_copy(src_hbm.at[pl.ds(base, _C)], sv, s0).wait()
        pltpu.make_async_copy(dst_hbm.at[pl.ds(base, _C)], dv, s1).wait()
        pltpu.make_async_copy(vals_hbm.at[pl.ds(base, _C)], vv, s2).wait()

    issue_idx(0, sets[0])
    wait_idx(0, sets[0])
    rbufs = (rowsA, rowsB)
    ssems = (ssA, ssB)

    def do_chunk(ci, p, first, st, st_next):
        sv, dv, vv = st[0], st[1], st[2]
        dv_o = st_next[1]
        rb, rb_o = rbufs[p], rbufs[1 - p]
        pltpu.async_copy(h_hbm.at[sv], rb, gsem).wait()
        scale(rb, vv)
        if first:

            @pl.when(ci > 0)
            def _():
                pltpu.make_async_copy(rb_o, acc_sh.at[dv_o], ssems[1 - p]
                                      ).wait()
        else:
            pltpu.make_async_copy(rb_o, acc_sh.at[dv_o], ssems[1 - p]).wait()
        issue_idx(ci + 1, st_next)
        pltpu.async_copy(rb, acc_sh.at[dv], ssems[p], add=True)
        wait_idx(ci + 1, st_next)

    def pair(i2, carry):
        a = 2 * i2
        do_chunk(a, 0, True, sets[0], sets[1])
        do_chunk(a + 1, 1, False, sets[1], sets[0])
        return carry

    lax.fori_loop(0, _NCH // 2, pair, 0)
    pltpu.make_async_copy(rowsB, acc_sh.at[dstB], ssB).wait()
    plsc.subcore_barrier()

    r0 = s * _RT
    pltpu.sync_copy(acc_sh.at[pl.ds(r0, _RT)], out_hbm.at[c, pl.ds(r0, _RT)])


def _mm_body(x_ref, w_ref, o_ref):
    o_ref[...] = jnp.dot(x_ref[...], w_ref[...],
                         preferred_element_type=jnp.float32)


def _mid_body(p_ref, b_ref, o_ref):
    agg = p_ref[0, :_N] + p_ref[1, :_N] + b_ref[...]
    col_mean = jnp.mean(agg, axis=0, keepdims=True)
    xc = agg - col_mean
    rownorm_mean = jnp.sqrt(1e-06 + jnp.mean(jnp.sum(xc * xc, axis=1)))
    o_ref[...] = jnp.maximum(_NORM_SCALE * xc / rownorm_mean, 0.0)


def _fin_body(p_ref, w_ref, b_ref, o_ref):
    agg = p_ref[0, :_N] + p_ref[1, :_N]
    o_ref[...] = jnp.dot(agg, w_ref[...],
                         preferred_element_type=jnp.float32) + b_ref[...]


def _pack_edges(edge_index, adj_values):
    pad_e = _ECP - _EW
    dst = edge_index[0].reshape(_NW, _EW)
    src = edge_index[1].reshape(_NW, _EW)
    vals = adj_values.reshape(_NW, _EW)
    pad_dst = jnp.broadcast_to(
        _N + (jnp.arange(pad_e, dtype=jnp.int32) % (_NP - _N)), (_NW, pad_e))
    zi = jnp.zeros((_NW, pad_e), jnp.int32)
    zf = jnp.zeros((_NW, pad_e), jnp.float32)
    dstp = jnp.concatenate([dst, pad_dst], axis=1).reshape(_NW * _ECP)
    srcp = jnp.concatenate([src, zi], axis=1).reshape(_NW * _ECP)
    valsp = jnp.concatenate([vals, zf], axis=1).reshape(_NW * _ECP)
    return srcp, dstp, valsp


def kernel(x, edge_index, adj_values, weight1, bias1, weight_out, bias_out):
    srcp, dstp, valsp = _pack_edges(edge_index, adj_values)
    h = pl.pallas_call(
        _mm_body,
        out_shape=jax.ShapeDtypeStruct((_N, _F), jnp.float32),
    )(x, weight1)
    p1 = _spmm(h, srcp, dstp, valsp)
    h2 = pl.pallas_call(
        _mid_body,
        out_shape=jax.ShapeDtypeStruct((_N, _F), jnp.float32),
    )(p1, bias1)
    p2 = _spmm(h2, srcp, dstp, valsp)
    out = pl.pallas_call(
        _fin_body,
        out_shape=jax.ShapeDtypeStruct((_N, 64), jnp.float32),
    )(p2, weight_out, bias_out)
    return out


# speedup vs baseline: 1.0050x; 1.0050x over previous
import functools

import jax
import jax.numpy as jnp
from jax import lax
from jax.experimental import pallas as pl
from jax.experimental.pallas import tpu as pltpu
from jax.experimental.pallas import tpu_sc as plsc

_N = 10000
_E = 320000
_F = 128
_NORM_SCALE = 1.0

_NC = 2
_NS = 16
_NW = _NC * _NS
_EW = _E // _NW
_C = 80
_ECP = 10240
_NCH = 126
_NP = 10240
_RT = _NP // _NS

_mesh = plsc.VectorSubcoreMesh(core_axis_name="c", subcore_axis_name="s")


@functools.partial(
    pl.kernel,
    mesh=_mesh,
    out_type=jax.ShapeDtypeStruct((_NC, _NP, _F), jnp.float32),
    scratch_types=(
        [pltpu.VMEM((_C, _F), jnp.float32)]
        + [pltpu.VMEM((_C,), jnp.int32) for _ in range(4)]
        + [pltpu.VMEM((_C,), jnp.float32) for _ in range(2)]
        + [pltpu.VMEM_SHARED((_NP, _F), jnp.float32)]
        + [pltpu.SemaphoreType.DMA for _ in range(7)]
    ),
)
def _spmm(h_hbm, src_hbm, dst_hbm, vals_hbm, out_hbm,
          rows, srcA, srcB, dstA, dstB, valsA, valsB, acc_sh,
          gsem, isA0, isA1, isA2, isB0, isB1, isB2):
    c = lax.axis_index("c")
    s = lax.axis_index("s")
    wid = c * _NS + s

    zvec = jnp.zeros((16,), jnp.float32)

    def zrow(r, carry):
        for j in range(_F // 16):
            rows[r, pl.ds(j * 16, 16)] = zvec
        return carry

    lax.fori_loop(0, _C, zrow, 0)
    for k in range(_RT // _C):
        pltpu.sync_copy(rows, acc_sh.at[pl.ds(s * _RT + k * _C, _C)])
    plsc.subcore_barrier()

    def scale(buf, vv):
        def group(g, gcarry):
            v16 = vv[pl.ds(g * 16, 16)]
            for i in range(16):
                vvec = v16[jnp.full((16,), i, jnp.int32)]
                r = g * 16 + i
                for j in range(_F // 16):
                    seg = buf[r, pl.ds(j * 16, 16)]
                    buf[r, pl.ds(j * 16, 16)] = seg * vvec
            return gcarry

        lax.fori_loop(0, _C // 16, group, 0)

    sets = ((srcA, dstA, valsA, isA0, isA1, isA2),
            (srcB, dstB, valsB, isB0, isB1, isB2))

    def issue_idx(m, st):
        sv, dv, vv, s0, s1, s2 = st
        base = wid * _ECP + m * _C
        pltpu.async_copy(src_hbm.at[pl.ds(base, _C)], sv, s0)
        pltpu.async_copy(dst_hbm.at[pl.ds(base, _C)], dv, s1)
        pltpu.async_copy(vals_hbm.at[pl.ds(base, _C)], vv, s2)

    def wait_idx(m, st):
        sv, dv, vv, s0, s1, s2 = st
        base = wid * _ECP + m * _C
        pltpu.make_async_copy(src_hbm.at[pl.ds(base, _C)], sv, s0).wait()
        pltpu.make_async_copy(dst_hbm.at[pl.ds(base, _C)], dv, s1).wait()
        pltpu.make_async_copy(vals_hbm.at[pl.ds(base, _C)], vv, s2).wait()

    issue_idx(0, sets[0])
    wait_idx(0, sets[0])

    def do_chunk(ci, st, st_next):
        sv, dv, vv = st[0], st[1], st[2]
        pltpu.async_copy(h_hbm.at[sv], rows, gsem).wait()
        issue_idx(ci + 1, st_next)
        scale(rows, vv)
        pltpu.sync_copy(rows, acc_sh.at[dv], add=True)

    def pair(i2, carry):
        a = 2 * i2
        do_chunk(a, sets[0], sets[1])
        wait_idx(a + 1, sets[1])
        do_chunk(a + 1, sets[1], sets[0])
        wait_idx(a + 2, sets[0])
        return carry

    lax.fori_loop(0, _NCH // 2, pair, 0)
    plsc.subcore_barrier()

    r0 = s * _RT
    pltpu.sync_copy(acc_sh.at[pl.ds(r0, _RT)], out_hbm.at[c, pl.ds(r0, _RT)])


def _mm_body(x_ref, w_ref, o_ref):
    o_ref[...] = jnp.dot(x_ref[...], w_ref[...],
                         preferred_element_type=jnp.float32)


def _mid_body(p_ref, b_ref, o_ref):
    agg = p_ref[0, :_N] + p_ref[1, :_N] + b_ref[...]
    col_mean = jnp.mean(agg, axis=0, keepdims=True)
    xc = agg - col_mean
    rownorm_mean = jnp.sqrt(1e-06 + jnp.mean(jnp.sum(xc * xc, axis=1)))
    o_ref[...] = jnp.maximum(_NORM_SCALE * xc / rownorm_mean, 0.0)


def _fin_body(p_ref, w_ref, b_ref, o_ref):
    agg = p_ref[0, :_N] + p_ref[1, :_N]
    o_ref[...] = jnp.dot(agg, w_ref[...],
                         preferred_element_type=jnp.float32) + b_ref[...]


def _pack_edges(edge_index, adj_values):
    pad_e = _ECP - _EW
    dst = edge_index[0].reshape(_NW, _EW)
    src = edge_index[1].reshape(_NW, _EW)
    vals = adj_values.reshape(_NW, _EW)
    pad_dst = jnp.broadcast_to(
        _N + (jnp.arange(pad_e, dtype=jnp.int32) % (_NP - _N)), (_NW, pad_e))
    zi = jnp.zeros((_NW, pad_e), jnp.int32)
    zf = jnp.zeros((_NW, pad_e), jnp.float32)
    dstp = jnp.concatenate([dst, pad_dst], axis=1).reshape(_NW * _ECP)
    srcp = jnp.concatenate([src, zi], axis=1).reshape(_NW * _ECP)
    valsp = jnp.concatenate([vals, zf], axis=1).reshape(_NW * _ECP)
    return srcp, dstp, valsp


def kernel(x, edge_index, adj_values, weight1, bias1, weight_out, bias_out):
    srcp, dstp, valsp = _pack_edges(edge_index, adj_values)
    h = pl.pallas_call(
        _mm_body,
        out_shape=jax.ShapeDtypeStruct((_N, _F), jnp.float32),
    )(x, weight1)
    p1 = _spmm(h, srcp, dstp, valsp)
    h2 = pl.pallas_call(
        _mid_body,
        out_shape=jax.ShapeDtypeStruct((_N, _F), jnp.float32),
    )(p1, bias1)
    p2 = _spmm(h2, srcp, dstp, valsp)
    out = pl.pallas_call(
        _fin_body,
        out_shape=jax.ShapeDtypeStruct((_N, 64), jnp.float32),
    )(p2, weight_out, bias_out)
    return out
